# baseline (speedup 1.0000x reference)
import functools

import numpy as np
import jax
import jax.numpy as jnp
from jax import lax
from jax.experimental import pallas as pl
from jax.experimental.pallas import tpu as pltpu
from jax.experimental.pallas import tpu_sc as plsc
from jax._src.random.threefry2x32 import threefry2x32_p

P = 8192
RS = 128
NSTEPS = P // RS
KT = P // 128

NC = 2
NS = 16
NW = NC * NS
BPW = P // NW

TINY = np.float32(np.finfo(np.float32).tiny)
SPAN = np.float32(np.float32(1.0) - TINY)
BIG = np.int32(2**30)


def _bits(k2_const, lin_u32):
    z = jnp.zeros_like(lin_u32)
    b1, b2 = threefry2x32_p.bind(
        jnp.uint32(0), jnp.uint32(k2_const), z, lin_u32)
    return b1 ^ b2


def _unit_float(bits):
    fb = lax.shift_right_logical(bits, jnp.uint32(9)) | jnp.uint32(0x3F800000)
    return lax.bitcast_convert_type(fb, jnp.float32) - jnp.float32(1.0)


def _body(logit_ref, idx_ref, log_scr):
    g = pl.program_id(0)

    @pl.when(g == 0)
    def _init():
        log_scr[...] = jnp.broadcast_to(logit_ref[...], (RS, P))

    rio = lax.broadcasted_iota(jnp.int32, (RS, 128), 0)
    cio2 = lax.broadcasted_iota(jnp.int32, (RS, 128), 1)
    lin0 = ((g * RS + rio) * P + cio2).astype(jnp.uint32)

    def k_body(k, carry):
        m, a = carry
        lin = lin0 + (k * 128).astype(jnp.uint32)
        f = _unit_float(_bits(2, lin))
        u = jnp.maximum(TINY, f * SPAN + TINY)
        lchunk = log_scr[:, pl.ds(k * 128, 128)]
        t = -jnp.log(-jnp.log(u)) + lchunk
        sel = t > m
        a = jnp.where(sel, k * 128 + cio2, a)
        m = jnp.where(sel, t, m)
        return m, a

    m, a = lax.fori_loop(
        0, KT, k_body,
        (jnp.full((RS, 128), -jnp.inf, jnp.float32),
         jnp.zeros((RS, 128), jnp.int32)),
        unroll=8)

    m_row = jnp.max(m, axis=1, keepdims=True)
    a_row = jnp.min(jnp.where(m == m_row, a, BIG),
                    axis=1, keepdims=True)
    idx_ref[...] = a_row


def _sc_gather(idx2d, upd128):
    mesh = plsc.VectorSubcoreMesh(core_axis_name="c", subcore_axis_name="s")

    @functools.partial(
        pl.kernel,
        out_type=jax.ShapeDtypeStruct((NW, 16), jnp.float32),
        mesh=mesh,
        scratch_types=[
            pltpu.VMEM((BPW // 128, 128), jnp.int32),
            pltpu.VMEM((128, 128), jnp.float32),
            pltpu.VMEM((16,), jnp.float32),
            pltpu.SemaphoreType.DMA,
        ],
    )
    def run(idx_hbm, upd_hbm, out_hbm, idx_v, rows_v, acc_v, sem):
        wid = lax.axis_index("s") * NC + lax.axis_index("c")
        nrows = BPW // 128
        pltpu.sync_copy(idx_hbm.at[pl.ds(wid * nrows, nrows)], idx_v)
        acc = jnp.zeros((16,), jnp.float32)
        for j in range(nrows):
            pltpu.async_copy(upd_hbm.at[idx_v.at[j]], rows_v, sem).wait()

            def body(i, acc):
                return acc + rows_v[i, pl.ds(0, 16)]

            acc = lax.fori_loop(0, 128, body, acc)
        acc_v[...] = acc
        pltpu.sync_copy(acc_v, out_hbm.at[wid])

    return run(idx2d, upd128)


def kernel(inputs, state_vector, transition_matrix, process_noise_cov,
           forward_matrix):
    updated = jnp.matmul(state_vector, transition_matrix.T)
    noise = jax.random.normal(jax.random.key(1), state_vector.shape,
                              dtype=jnp.float32)
    chol = jnp.linalg.cholesky(process_noise_cov)
    updated = updated + jnp.matmul(noise, chol)
    predicted = jnp.matmul(forward_matrix, updated.reshape(-1, 3).T)
    diff = inputs.reshape(1, 1, P) - predicted.T
    w = jnp.sum(jnp.square(diff), axis=-1).reshape(P)
    logits = jnp.log(w)

    idx = pl.pallas_call(
        _body,
        grid=(NSTEPS,),
        in_specs=[pl.BlockSpec((1, P), lambda g: (0, 0))],
        out_specs=pl.BlockSpec((RS, 1), lambda g: (g, 0)),
        out_shape=jax.ShapeDtypeStruct((P, 1), jnp.int32),
        scratch_shapes=[pltpu.VMEM((RS, P), jnp.float32)],
        compiler_params=pltpu.CompilerParams(
            dimension_semantics=("arbitrary",)),
    )(logits.reshape(1, P))

    upd128 = jnp.pad(updated, ((0, 0), (0, 125)))
    partials = _sc_gather(idx.reshape(P // 128, 128), upd128)
    total = jnp.sum(partials, axis=0)
    return total[:3] / np.float32(P)


# speedup vs baseline: 1.0287x; 1.0287x over previous
import functools

import numpy as np
import jax
import jax.numpy as jnp
from jax import lax
from jax.experimental import pallas as pl
from jax.experimental.pallas import tpu as pltpu
from jax.experimental.pallas import tpu_sc as plsc
from jax._src.random.threefry2x32 import threefry2x32_p

P = 8192
RS = 64
NSTEPS = P // RS
KT = P // 128

NC = 2
NS = 16
NW = NC * NS
BPW = P // NW

TINY = np.float32(np.finfo(np.float32).tiny)
SPAN = np.float32(np.float32(1.0) - TINY)
BIG = np.int32(2**30)


def _bits(k2_const, lin_u32):
    z = jnp.zeros_like(lin_u32)
    b1, b2 = threefry2x32_p.bind(
        jnp.uint32(0), jnp.uint32(k2_const), z, lin_u32)
    return b1 ^ b2


def _unit_float(bits):
    fb = lax.shift_right_logical(bits, jnp.uint32(9)) | jnp.uint32(0x3F800000)
    return lax.bitcast_convert_type(fb, jnp.float32) - jnp.float32(1.0)


def _body(logit_ref, idx_ref, log_scr):
    g = pl.program_id(0)

    @pl.when(g == 0)
    def _init():
        log_scr[...] = jnp.broadcast_to(logit_ref[...], (RS, P))

    rio = lax.broadcasted_iota(jnp.int32, (RS, 128), 0)
    cio2 = lax.broadcasted_iota(jnp.int32, (RS, 128), 1)
    lin0 = ((g * RS + rio) * P + cio2).astype(jnp.uint32)

    def k_body(k, carry):
        m, a = carry
        lin = lin0 + (k * 128).astype(jnp.uint32)
        f = _unit_float(_bits(2, lin))
        u = jnp.maximum(TINY, f * SPAN + TINY)
        lchunk = log_scr[:, pl.ds(k * 128, 128)]
        t = -jnp.log(-jnp.log(u)) + lchunk
        sel = t > m
        a = jnp.where(sel, k * 128 + cio2, a)
        m = jnp.where(sel, t, m)
        return m, a

    m, a = lax.fori_loop(
        0, KT, k_body,
        (jnp.full((RS, 128), -jnp.inf, jnp.float32),
         jnp.zeros((RS, 128), jnp.int32)),
        unroll=8)

    m_row = jnp.max(m, axis=1, keepdims=True)
    a_row = jnp.min(jnp.where(m == m_row, a, BIG),
                    axis=1, keepdims=True)
    idx_ref[...] = a_row


def _sc_gather(idx2d, upd128):
    mesh = plsc.VectorSubcoreMesh(core_axis_name="c", subcore_axis_name="s")

    @functools.partial(
        pl.kernel,
        out_type=jax.ShapeDtypeStruct((NW, 16), jnp.float32),
        mesh=mesh,
        scratch_types=[
            pltpu.VMEM((BPW // 128, 128), jnp.int32),
            pltpu.VMEM((128, 128), jnp.float32),
            pltpu.VMEM((16,), jnp.float32),
            pltpu.SemaphoreType.DMA,
        ],
    )
    def run(idx_hbm, upd_hbm, out_hbm, idx_v, rows_v, acc_v, sem):
        wid = lax.axis_index("s") * NC + lax.axis_index("c")
        nrows = BPW // 128
        pltpu.sync_copy(idx_hbm.at[pl.ds(wid * nrows, nrows)], idx_v)
        acc = jnp.zeros((16,), jnp.float32)
        for j in range(nrows):
            pltpu.async_copy(upd_hbm.at[idx_v.at[j]], rows_v, sem).wait()

            def body(i, acc):
                return acc + rows_v[i, pl.ds(0, 16)]

            acc = lax.fori_loop(0, 128, body, acc)
        acc_v[...] = acc
        pltpu.sync_copy(acc_v, out_hbm.at[wid])

    return run(idx2d, upd128)


def kernel(inputs, state_vector, transition_matrix, process_noise_cov,
           forward_matrix):
    updated = jnp.matmul(state_vector, transition_matrix.T)
    noise = jax.random.normal(jax.random.key(1), state_vector.shape,
                              dtype=jnp.float32)
    chol = jnp.linalg.cholesky(process_noise_cov)
    updated = updated + jnp.matmul(noise, chol)
    predicted = jnp.matmul(forward_matrix, updated.reshape(-1, 3).T)
    diff = inputs.reshape(1, 1, P) - predicted.T
    w = jnp.sum(jnp.square(diff), axis=-1).reshape(P)
    logits = jnp.log(w)

    idx = pl.pallas_call(
        _body,
        grid=(NSTEPS,),
        in_specs=[pl.BlockSpec((1, P), lambda g: (0, 0))],
        out_specs=pl.BlockSpec((RS, 1), lambda g: (g, 0)),
        out_shape=jax.ShapeDtypeStruct((P, 1), jnp.int32),
        scratch_shapes=[pltpu.VMEM((RS, P), jnp.float32)],
        compiler_params=pltpu.CompilerParams(
            dimension_semantics=("arbitrary",)),
    )(logits.reshape(1, P))

    upd128 = jnp.pad(updated, ((0, 0), (0, 125)))
    partials = _sc_gather(idx.reshape(P // 128, 128), upd128)
    total = jnp.sum(partials, axis=0)
    return total[:3] / np.float32(P)
